# Initial kernel scaffold; baseline (speedup 1.0000x reference)
import jax
import jax.numpy as jnp
from jax.experimental import pallas as pl


def kernel(hidden_states, W):
    raise NotImplementedError("write your pallas kernel here")


# speedup vs baseline: 1.4538x; 1.4538x over previous
import jax
import jax.numpy as jnp
from jax import lax
from jax.experimental import pallas as pl
from jax.experimental.pallas import tpu as pltpu

_NUM_EXPERTS = 64
_HIDDEN = 1024
_TOP_K = 2
_BT = 512


def _router_body(x_ref, w_ref, wout_ref, logits_ref, iout_ref):
    x = x_ref[...]
    w = w_ref[...]
    logits = lax.dot_general(
        x, w, (((1,), (1,)), ((), ())), preferred_element_type=jnp.float32
    )
    logits_ref[...] = logits

    e_iota = lax.broadcasted_iota(jnp.int32, logits.shape, 1)
    m1 = jnp.max(logits, axis=1, keepdims=True)
    i1 = jnp.min(jnp.where(logits == m1, e_iota, _NUM_EXPERTS), axis=1,
                 keepdims=True)
    masked = jnp.where(e_iota == i1, -jnp.inf, logits)
    m2 = jnp.max(masked, axis=1, keepdims=True)
    i2 = jnp.min(jnp.where(masked == m2, e_iota, _NUM_EXPERTS), axis=1,
                 keepdims=True)

    e2 = jnp.exp(m2 - m1)
    s = 1.0 + e2
    w1 = 1.0 / s
    w2 = e2 / s
    wout_ref[...] = jnp.concatenate([w1, w2], axis=1)
    iout_ref[...] = jnp.concatenate([i1, i2], axis=1)


def kernel(hidden_states, W):
    T, H = hidden_states.shape
    E = W.shape[0]
    grid = (T // _BT,)
    wout, logits, iout = pl.pallas_call(
        _router_body,
        grid=grid,
        in_specs=[
            pl.BlockSpec((_BT, H), lambda i: (i, 0)),
            pl.BlockSpec((E, H), lambda i: (0, 0)),
        ],
        out_specs=[
            pl.BlockSpec((_BT, _TOP_K), lambda i: (i, 0)),
            pl.BlockSpec((_BT, E), lambda i: (i, 0)),
            pl.BlockSpec((_BT, _TOP_K), lambda i: (i, 0)),
        ],
        out_shape=[
            jax.ShapeDtypeStruct((T, _TOP_K), jnp.float32),
            jax.ShapeDtypeStruct((T, E), jnp.float32),
            jax.ShapeDtypeStruct((T, _TOP_K), jnp.int32),
        ],
    )(hidden_states, W)
    return (wout, logits, iout)


# speedup vs baseline: 1.8358x; 1.2628x over previous
import jax
import jax.numpy as jnp
from jax import lax
from jax.experimental import pallas as pl
from jax.experimental.pallas import tpu as pltpu

_NUM_EXPERTS = 64
_HIDDEN = 1024
_TOP_K = 2
_BT = 1024


def _router_body(x_ref, w_ref, wout_ref, logits_ref, iout_ref):
    x = x_ref[...]
    w = w_ref[...]
    logits = lax.dot_general(
        x, w, (((1,), (1,)), ((), ())), preferred_element_type=jnp.float32
    )
    logits_ref[...] = logits

    e_iota = lax.broadcasted_iota(jnp.int32, logits.shape, 1)
    m1 = jnp.max(logits, axis=1, keepdims=True)
    i1 = jnp.min(jnp.where(logits == m1, e_iota, _NUM_EXPERTS), axis=1,
                 keepdims=True)
    masked = jnp.where(e_iota == i1, -jnp.inf, logits)
    m2 = jnp.max(masked, axis=1, keepdims=True)
    i2 = jnp.min(jnp.where(masked == m2, e_iota, _NUM_EXPERTS), axis=1,
                 keepdims=True)

    e2 = jnp.exp(m2 - m1)
    s = 1.0 + e2
    w1 = 1.0 / s
    w2 = e2 / s
    wout_ref[...] = jnp.concatenate([w1, w2], axis=1)
    iout_ref[...] = jnp.concatenate([i1, i2], axis=1)


def kernel(hidden_states, W):
    T, H = hidden_states.shape
    E = W.shape[0]
    grid = (T // _BT,)
    wout, logits, iout = pl.pallas_call(
        _router_body,
        grid=grid,
        in_specs=[
            pl.BlockSpec((_BT, H), lambda i: (i, 0)),
            pl.BlockSpec((E, H), lambda i: (0, 0)),
        ],
        out_specs=[
            pl.BlockSpec((_BT, _TOP_K), lambda i: (i, 0)),
            pl.BlockSpec((_BT, E), lambda i: (i, 0)),
            pl.BlockSpec((_BT, _TOP_K), lambda i: (i, 0)),
        ],
        out_shape=[
            jax.ShapeDtypeStruct((T, _TOP_K), jnp.float32),
            jax.ShapeDtypeStruct((T, E), jnp.float32),
            jax.ShapeDtypeStruct((T, _TOP_K), jnp.int32),
        ],
    )(hidden_states, W)
    return (wout, logits, iout)


# speedup vs baseline: 1.9648x; 1.0703x over previous
import jax
import jax.numpy as jnp
from jax import lax
from jax.experimental import pallas as pl
from jax.experimental.pallas import tpu as pltpu

_NUM_EXPERTS = 64
_HIDDEN = 1024
_TOP_K = 2
_BT = 2048


def _router_body(x_ref, w_ref, wout_ref, logits_ref, iout_ref):
    x = x_ref[...]
    w = w_ref[...]
    logits = lax.dot_general(
        x, w, (((1,), (1,)), ((), ())), preferred_element_type=jnp.float32
    )
    logits_ref[...] = logits

    e_iota = lax.broadcasted_iota(jnp.int32, logits.shape, 1)
    m1 = jnp.max(logits, axis=1, keepdims=True)
    i1 = jnp.min(jnp.where(logits == m1, e_iota, _NUM_EXPERTS), axis=1,
                 keepdims=True)
    masked = jnp.where(e_iota == i1, -jnp.inf, logits)
    m2 = jnp.max(masked, axis=1, keepdims=True)
    i2 = jnp.min(jnp.where(masked == m2, e_iota, _NUM_EXPERTS), axis=1,
                 keepdims=True)

    e2 = jnp.exp(m2 - m1)
    s = 1.0 + e2
    w1 = 1.0 / s
    w2 = e2 / s
    wout_ref[...] = jnp.concatenate([w1, w2], axis=1)
    iout_ref[...] = jnp.concatenate([i1, i2], axis=1)


def kernel(hidden_states, W):
    T, H = hidden_states.shape
    E = W.shape[0]
    grid = (T // _BT,)
    wout, logits, iout = pl.pallas_call(
        _router_body,
        grid=grid,
        in_specs=[
            pl.BlockSpec((_BT, H), lambda i: (i, 0)),
            pl.BlockSpec((E, H), lambda i: (0, 0)),
        ],
        out_specs=[
            pl.BlockSpec((_BT, _TOP_K), lambda i: (i, 0)),
            pl.BlockSpec((_BT, E), lambda i: (i, 0)),
            pl.BlockSpec((_BT, _TOP_K), lambda i: (i, 0)),
        ],
        out_shape=[
            jax.ShapeDtypeStruct((T, _TOP_K), jnp.float32),
            jax.ShapeDtypeStruct((T, E), jnp.float32),
            jax.ShapeDtypeStruct((T, _TOP_K), jnp.int32),
        ],
    )(hidden_states, W)
    return (wout, logits, iout)


# speedup vs baseline: 2.0647x; 1.0508x over previous
import jax
import jax.numpy as jnp
from jax import lax
from jax.experimental import pallas as pl
from jax.experimental.pallas import tpu as pltpu

_NUM_EXPERTS = 64
_HIDDEN = 1024
_TOP_K = 2
_BT = 4096


def _router_body(x_ref, w_ref, wout_ref, logits_ref, iout_ref):
    x = x_ref[...]
    w = w_ref[...]
    logits = lax.dot_general(
        x, w, (((1,), (1,)), ((), ())), preferred_element_type=jnp.float32
    )
    logits_ref[...] = logits

    e_iota = lax.broadcasted_iota(jnp.int32, logits.shape, 1)
    m1 = jnp.max(logits, axis=1, keepdims=True)
    i1 = jnp.min(jnp.where(logits == m1, e_iota, _NUM_EXPERTS), axis=1,
                 keepdims=True)
    masked = jnp.where(e_iota == i1, -jnp.inf, logits)
    m2 = jnp.max(masked, axis=1, keepdims=True)
    i2 = jnp.min(jnp.where(masked == m2, e_iota, _NUM_EXPERTS), axis=1,
                 keepdims=True)

    e2 = jnp.exp(m2 - m1)
    s = 1.0 + e2
    w1 = 1.0 / s
    w2 = e2 / s
    wout_ref[...] = jnp.concatenate([w1, w2], axis=1)
    iout_ref[...] = jnp.concatenate([i1, i2], axis=1)


def kernel(hidden_states, W):
    T, H = hidden_states.shape
    E = W.shape[0]
    grid = (T // _BT,)
    wout, logits, iout = pl.pallas_call(
        _router_body,
        grid=grid,
        in_specs=[
            pl.BlockSpec((_BT, H), lambda i: (i, 0)),
            pl.BlockSpec((E, H), lambda i: (0, 0)),
        ],
        out_specs=[
            pl.BlockSpec((_BT, _TOP_K), lambda i: (i, 0)),
            pl.BlockSpec((_BT, E), lambda i: (i, 0)),
            pl.BlockSpec((_BT, _TOP_K), lambda i: (i, 0)),
        ],
        out_shape=[
            jax.ShapeDtypeStruct((T, _TOP_K), jnp.float32),
            jax.ShapeDtypeStruct((T, E), jnp.float32),
            jax.ShapeDtypeStruct((T, _TOP_K), jnp.int32),
        ],
    )(hidden_states, W)
    return (wout, logits, iout)
